# baseline (speedup 1.0000x reference)
import functools

import jax
import jax.numpy as jnp
from jax.experimental import pallas as pl


def _tree_rowmin(t):
    w = t.shape[1]
    while w > 128:
        w //= 2
        t = jnp.minimum(t[:, :w], t[:, w:])
    return t


def _tree_colmin8(t):
    h = t.shape[0]
    while h > 8:
        h //= 2
        t = jnp.minimum(t[:h], t[h:])
    return t


def _chamfer_body(a_ref, bt_ref, d1_ref, d2_ref, *, bn: int, mc: int):
    i = pl.program_id(1)
    ab = a_ref[0]
    bb = bt_ref[0]
    M = bb.shape[1]

    rowparts = None
    colparts = []
    for c in range(M // mc):
        dc = jax.lax.dot_general(
            ab, bb[:, c * mc : (c + 1) * mc],
            (((1,), (0,)), ((), ())),
            preferred_element_type=jnp.float32,
        )
        rp = _tree_rowmin(dc)
        rowparts = rp if rowparts is None else jnp.minimum(rowparts, rp)
        colparts.append(_tree_colmin8(dc))

    rt = rowparts.T
    d1_ref[0, 0, pl.ds(i * bn, bn)] = jnp.min(_tree_colmin8(rt), axis=0)
    cm = jnp.min(jnp.concatenate(colparts, axis=1), axis=0)

    @pl.when(i == 0)
    def _init():
        d2_ref[0, 0, :] = cm

    @pl.when(i > 0)
    def _acc():
        d2_ref[0, 0, :] = jnp.minimum(d2_ref[0, 0, :], cm)


def _hi_lo(x):
    hi = x.astype(jnp.bfloat16)
    lo = (x - hi.astype(jnp.float32)).astype(jnp.bfloat16)
    return hi, lo


@jax.jit
def kernel(input1, input2):
    B, N, _ = input1.shape
    _, M, _ = input2.shape
    bn = 512

    sq1 = jnp.sum(input1 * input1, axis=-1)
    sq2 = jnp.sum(input2 * input2, axis=-1)
    h1, l1 = _hi_lo(sq1)
    h2, l2 = _hi_lo(sq2)
    ones_n = jnp.ones((B, N), jnp.bfloat16)
    ones_m = jnp.ones((B, M), jnp.bfloat16)
    zer_n = jnp.zeros((B, N), jnp.bfloat16)
    zer_m = jnp.zeros((B, M), jnp.bfloat16)

    x1b = (-2.0 * input1).astype(jnp.bfloat16)
    x2b = input2.astype(jnp.bfloat16)

    aug1 = jnp.concatenate(
        [x1b,
         h1[..., None], l1[..., None],
         ones_n[..., None], ones_n[..., None],
         zer_n[..., None]], axis=-1)
    aug2 = jnp.concatenate(
        [x2b,
         ones_m[..., None], ones_m[..., None],
         h2[..., None], l2[..., None],
         zer_m[..., None]], axis=-1)
    aug2t = aug2.transpose(0, 2, 1)

    d1, d2 = pl.pallas_call(
        functools.partial(_chamfer_body, bn=bn, mc=1024),
        grid=(B, N // bn),
        in_specs=[
            pl.BlockSpec((1, bn, 8), lambda b, i: (b, i, 0)),
            pl.BlockSpec((1, 8, M), lambda b, i: (b, 0, 0)),
        ],
        out_specs=[
            pl.BlockSpec((1, 1, N), lambda b, i: (b, 0, 0)),
            pl.BlockSpec((1, 1, M), lambda b, i: (b, 0, 0)),
        ],
        out_shape=[
            jax.ShapeDtypeStruct((B, 1, N), jnp.float32),
            jax.ShapeDtypeStruct((B, 1, M), jnp.float32),
        ],
    )(aug1, aug2t)
    return d1.reshape(B, N), d2.reshape(B, M)


# speedup vs baseline: 1.6906x; 1.6906x over previous
import functools

import jax
import jax.numpy as jnp
from jax.experimental import pallas as pl


def _tree_rowmin(t):
    w = t.shape[1]
    while w > 128:
        w //= 2
        t = jnp.minimum(t[:, :w], t[:, w:])
    return t


def _tree_colmin8(t):
    h = t.shape[0]
    while h > 8:
        h //= 2
        t = jnp.minimum(t[:h], t[h:])
    return t


def _hi_lo(x):
    hi = x.astype(jnp.bfloat16)
    lo = (x - hi.astype(jnp.float32)).astype(jnp.bfloat16)
    return hi, lo


def _chamfer_body(x1_ref, x2t_ref, d1_ref, d2_ref, *, bn: int, mc: int):
    i = pl.program_id(1)
    x1b = x1_ref[0]
    x2b = x2t_ref[0]
    M = x2b.shape[1]

    sq1 = jnp.sum(x1b * x1b, axis=1, keepdims=True)
    sq2 = jnp.sum(x2b * x2b, axis=0, keepdims=True)
    h1, l1 = _hi_lo(sq1)
    h2, l2 = _hi_lo(sq2)

    z1 = jnp.zeros((bn, 1), jnp.bfloat16)
    o1 = jnp.ones((bn, 1), jnp.bfloat16)
    z2 = jnp.zeros((1, M), jnp.bfloat16)
    o2 = jnp.ones((1, M), jnp.bfloat16)

    aug1 = jnp.concatenate(
        [(-2.0 * x1b).astype(jnp.bfloat16), z1, h1, z1, l1, z1, o1, z1, o1, z1],
        axis=1)
    aug2 = jnp.concatenate(
        [x2b.astype(jnp.bfloat16), z2, o2, z2, o2, z2, h2, z2, l2, z2],
        axis=0)

    rowparts = None
    colparts = []
    for c in range(M // mc):
        dc = jax.lax.dot_general(
            aug1, aug2[:, c * mc : (c + 1) * mc],
            (((1,), (0,)), ((), ())),
            preferred_element_type=jnp.float32,
        )
        rp = _tree_rowmin(dc)
        rowparts = rp if rowparts is None else jnp.minimum(rowparts, rp)
        colparts.append(_tree_colmin8(dc))

    rt = rowparts.T
    d1_ref[0, 0, pl.ds(i * bn, bn)] = jnp.min(_tree_colmin8(rt), axis=0)
    cm = jnp.min(jnp.concatenate(colparts, axis=1), axis=0)

    @pl.when(i == 0)
    def _init():
        d2_ref[0, 0, :] = cm

    @pl.when(i > 0)
    def _acc():
        d2_ref[0, 0, :] = jnp.minimum(d2_ref[0, 0, :], cm)


@jax.jit
def kernel(input1, input2):
    B, N, _ = input1.shape
    _, M, _ = input2.shape
    bn = 512
    x2t = input2.transpose(0, 2, 1)

    d1, d2 = pl.pallas_call(
        functools.partial(_chamfer_body, bn=bn, mc=1024),
        grid=(B, N // bn),
        in_specs=[
            pl.BlockSpec((1, bn, 3), lambda b, i: (b, i, 0)),
            pl.BlockSpec((1, 3, M), lambda b, i: (b, 0, 0)),
        ],
        out_specs=[
            pl.BlockSpec((1, 1, N), lambda b, i: (b, 0, 0)),
            pl.BlockSpec((1, 1, M), lambda b, i: (b, 0, 0)),
        ],
        out_shape=[
            jax.ShapeDtypeStruct((B, 1, N), jnp.float32),
            jax.ShapeDtypeStruct((B, 1, M), jnp.float32),
        ],
    )(input1, x2t)
    return d1.reshape(B, N), d2.reshape(B, M)
